# baseline (speedup 1.0000x reference)
import functools
import math

import jax
import jax.numpy as jnp
from jax import lax
from jax.experimental import pallas as pl
from jax.experimental.pallas import tpu as pltpu
from jax.experimental.pallas import tpu_sc as plsc

D_MODEL = 64
SCALE = math.sqrt(D_MODEL)
NUM_CORES = 2
NUM_SUBCORES = 16
NW = NUM_CORES * NUM_SUBCORES


def _sc_embed(x, table2):
    S, T = x.shape
    rows_pw = S // NW
    n_groups = rows_pw
    n_blocks = rows_pw // 8
    mesh = plsc.VectorSubcoreMesh(
        core_axis_name="c", subcore_axis_name="s", num_cores=NUM_CORES
    )

    nchunk = (T + 15) // 16
    starts = [min(16 * c, T - 16) for c in range(nchunk)]

    @functools.partial(
        pl.kernel,
        out_type=jax.ShapeDtypeStruct((S, T, D_MODEL), jnp.float32),
        mesh=mesh,
        scratch_types=[
            [pltpu.VMEM((8, T), jnp.int32) for _ in range(2)],
            [pltpu.VMEM((T, 2 * D_MODEL), jnp.float32) for _ in range(2)],
            [pltpu.VMEM((T, D_MODEL), jnp.float32) for _ in range(2)],
            [pltpu.VMEM((T,), jnp.int32) for _ in range(2)],
            [pltpu.VMEM((T,), jnp.int32) for _ in range(2)],
            [pltpu.SemaphoreType.DMA for _ in range(2)],
            [pltpu.SemaphoreType.DMA for _ in range(2)],
            [pltpu.SemaphoreType.DMA for _ in range(2)],
        ],
        compiler_params=pltpu.CompilerParams(use_tc_tiling_on_sc=True),
    )
    def k(idx_hbm, table_hbm, out_hbm, idx8, gbufs, wbufs, qrows, prows,
          gsems, wsems, bsems):
        wid = lax.axis_index("s") * NUM_CORES + lax.axis_index("c")
        row0 = wid * rows_pw
        pltpu.sync_copy(idx_hbm.at[pl.ds(row0, 8)], idx8[0])

        def prep(vv, b):
            r8 = lax.rem(vv, 8)
            for s8 in range(2):
                @pl.when(lax.rem(lax.div(vv, 8), 2) == s8)
                def _s():
                    @pl.when((r8 == 0) & (vv >= 8))
                    def _wait_blk():
                        pltpu.make_async_copy(
                            idx_hbm.at[pl.ds(row0, 8)], idx8[s8], bsems[s8]
                        ).wait()
                    for st in starts:
                        sl = pl.ds(st, 16)
                        row = idx8[s8][r8, sl]
                        qrows[b][sl] = lax.shift_right_logical(row, 1)
                        prows[b][sl] = lax.mul(lax.rem(row, 2), 64)
            pltpu.async_copy(table_hbm.at[qrows[b]], gbufs[b], gsems[b])

        prep(0, 0)

        def work(v, bs):
            pltpu.make_async_copy(
                table_hbm.at[qrows[bs]], gbufs[bs], gsems[bs]
            ).wait()
            @pl.when(v >= 2)
            def _wait_w():
                pltpu.make_async_copy(
                    wbufs[bs], out_hbm.at[row0], wsems[bs]
                ).wait()

            def do_block(base):
                p64 = prows[bs][pl.ds(base, 16)]
                offs = [pl.multiple_of(p64[kk], 64) for kk in range(16)]
                for kk in range(16):
                    rowref = gbufs[bs].at[base + kk, pl.ds(offs[kk], 64)]
                    for c in range(D_MODEL // 16):
                        wbufs[bs][base + kk, pl.ds(16 * c, 16)] = rowref[
                            pl.ds(16 * c, 16)
                        ]

            def blk_body(blk, c2):
                do_block(pl.multiple_of(16 * blk, 16))
                return c2

            lax.fori_loop(0, T // 16, blk_body, 0)
            if T % 16:
                do_block(T - 16)

            pltpu.async_copy(wbufs[bs], out_hbm.at[row0 + v], wsems[bs])

        def visit2(i, carry):
            v0 = 2 * i
            @pl.when(lax.rem(v0, 8) == 0)
            def _pf():
                m1 = lax.div(v0, 8) + 1
                @pl.when(m1 < n_blocks)
                def _m():
                    for s8 in range(2):
                        @pl.when(lax.rem(m1, 2) == s8)
                        def _c():
                            pltpu.async_copy(
                                idx_hbm.at[pl.ds(row0 + m1 * 8, 8)],
                                idx8[s8],
                                bsems[s8],
                            )

            prep(v0 + 1, 1)
            work(v0, 0)
            @pl.when(v0 + 2 < n_groups)
            def _p0():
                prep(v0 + 2, 0)
            work(v0 + 1, 1)
            return carry

        lax.fori_loop(0, n_groups // 2, visit2, 0)

        for bs in range(2):
            pltpu.make_async_copy(
                wbufs[bs], out_hbm.at[row0], wsems[bs]
            ).wait()

    return k(x, table2)


def kernel(x, table):
    S, T = x.shape
    V = table.shape[0]
    table2 = table.reshape(V // 2, 2 * D_MODEL) * jnp.float32(SCALE)
    return _sc_embed(x.astype(jnp.int32), table2)


# speedup vs baseline: 1.0207x; 1.0207x over previous
import functools
import math

import jax
import jax.numpy as jnp
from jax import lax
from jax.experimental import pallas as pl
from jax.experimental.pallas import tpu as pltpu
from jax.experimental.pallas import tpu_sc as plsc

D_MODEL = 64
SCALE = math.sqrt(D_MODEL)
NUM_CORES = 2
NUM_SUBCORES = 16
NW = NUM_CORES * NUM_SUBCORES


def _sc_embed(x, table2):
    S, T = x.shape
    rows_pw = S // NW
    n_groups = rows_pw
    n_blocks = rows_pw // 8
    mesh = plsc.VectorSubcoreMesh(
        core_axis_name="c", subcore_axis_name="s", num_cores=NUM_CORES
    )

    nchunk = (T + 15) // 16
    starts = [min(16 * c, T - 16) for c in range(nchunk)]

    @functools.partial(
        pl.kernel,
        out_type=jax.ShapeDtypeStruct((S, T, D_MODEL), jnp.float32),
        mesh=mesh,
        scratch_types=[
            [pltpu.VMEM((8, T), jnp.int32) for _ in range(2)],
            [pltpu.VMEM((T, 2 * D_MODEL), jnp.float32) for _ in range(2)],
            [pltpu.VMEM((T, D_MODEL), jnp.float32) for _ in range(2)],
            [pltpu.VMEM((T,), jnp.int32) for _ in range(2)],
            [pltpu.VMEM((T,), jnp.int32) for _ in range(2)],
            [pltpu.SemaphoreType.DMA for _ in range(2)],
            [pltpu.SemaphoreType.DMA for _ in range(2)],
            [pltpu.SemaphoreType.DMA for _ in range(2)],
        ],
        compiler_params=pltpu.CompilerParams(use_tc_tiling_on_sc=True),
    )
    def k(idx_hbm, table_hbm, out_hbm, idx8, gbufs, wbufs, qrows, prows,
          gsems, wsems, bsems):
        wid = lax.axis_index("s") * NUM_CORES + lax.axis_index("c")
        row0 = wid * rows_pw
        pltpu.sync_copy(idx_hbm.at[pl.ds(row0, 8)], idx8[0])

        def prep(vv, b):
            r8 = lax.rem(vv, 8)
            for s8 in range(2):
                @pl.when(lax.rem(lax.div(vv, 8), 2) == s8)
                def _s():
                    @pl.when((r8 == 0) & (vv >= 8))
                    def _wait_blk():
                        pltpu.make_async_copy(
                            idx_hbm.at[pl.ds(row0, 8)], idx8[s8], bsems[s8]
                        ).wait()
                    for st in starts:
                        sl = pl.ds(st, 16)
                        row = idx8[s8][r8, sl]
                        qrows[b][sl] = lax.shift_right_logical(row, 1)
                        prows[b][sl] = lax.mul(lax.rem(row, 2), 64)
            pltpu.async_copy(table_hbm.at[qrows[b]], gbufs[b], gsems[b])

        prep(0, 0)

        def work(v, bs):
            pltpu.make_async_copy(
                table_hbm.at[qrows[bs]], gbufs[bs], gsems[bs]
            ).wait()
            @pl.when(v >= 2)
            def _wait_w():
                pltpu.make_async_copy(
                    wbufs[bs], out_hbm.at[row0], wsems[bs]
                ).wait()

            def do_block(base):
                p64 = prows[bs][pl.ds(base, 16)]
                offs = [pl.multiple_of(p64[kk], 64) for kk in range(16)]
                for kk in range(16):
                    rowref = gbufs[bs].at[base + kk, pl.ds(offs[kk], 64)]
                    for c in range(D_MODEL // 16):
                        wbufs[bs][base + kk, pl.ds(16 * c, 16)] = (
                            rowref[pl.ds(16 * c, 16)] * SCALE
                        )

            def blk_body(blk, c2):
                do_block(pl.multiple_of(16 * blk, 16))
                return c2

            lax.fori_loop(0, T // 16, blk_body, 0)
            if T % 16:
                do_block(T - 16)

            pltpu.async_copy(wbufs[bs], out_hbm.at[row0 + v], wsems[bs])

        def visit2(i, carry):
            v0 = 2 * i
            @pl.when(lax.rem(v0, 8) == 0)
            def _pf():
                m1 = lax.div(v0, 8) + 1
                @pl.when(m1 < n_blocks)
                def _m():
                    for s8 in range(2):
                        @pl.when(lax.rem(m1, 2) == s8)
                        def _c():
                            pltpu.async_copy(
                                idx_hbm.at[pl.ds(row0 + m1 * 8, 8)],
                                idx8[s8],
                                bsems[s8],
                            )

            prep(v0 + 1, 1)
            work(v0, 0)
            @pl.when(v0 + 2 < n_groups)
            def _p0():
                prep(v0 + 2, 0)
            work(v0 + 1, 1)
            return carry

        lax.fori_loop(0, n_groups // 2, visit2, 0)

        for bs in range(2):
            pltpu.make_async_copy(
                wbufs[bs], out_hbm.at[row0], wsems[bs]
            ).wait()

    return k(x, table2)


def kernel(x, table):
    S, T = x.shape
    V = table.shape[0]
    table2 = table.reshape(V // 2, 2 * D_MODEL)
    return _sc_embed(x.astype(jnp.int32), table2)
